# Initial kernel scaffold; baseline (speedup 1.0000x reference)
import jax
import jax.numpy as jnp
from jax.experimental import pallas as pl


def kernel(x, position_ids, cos_cached, sin_cached):
    raise NotImplementedError("write your pallas kernel here")


# speedup vs baseline: 1.6253x; 1.6253x over previous
import functools

import jax
import jax.numpy as jnp
from jax import lax
from jax.experimental import pallas as pl
from jax.experimental.pallas import tpu as pltpu
from jax.experimental.pallas import tpu_sc as plsc

HEAD_DIM = 256
B_TOTAL = 4 * 8192

_info = plsc.get_sparse_core_info()
_NC, _NS = _info.num_cores, _info.num_subcores
_NW = _NC * _NS
_B_PER_W = B_TOTAL // _NW
_CHUNK = 128
_NCHUNK = _B_PER_W // _CHUNK


def _rope_gather(pos_flat, cos_cached, sin_cached):
    mesh = plsc.VectorSubcoreMesh(core_axis_name="c", subcore_axis_name="s")

    @functools.partial(
        pl.kernel,
        mesh=mesh,
        out_type=[
            jax.ShapeDtypeStruct((B_TOTAL, HEAD_DIM), jnp.float32),
            jax.ShapeDtypeStruct((B_TOTAL, HEAD_DIM), jnp.float32),
        ],
        scratch_types=[
            pltpu.VMEM((_B_PER_W,), jnp.int32),
            pltpu.VMEM((_CHUNK, HEAD_DIM), jnp.float32),
            pltpu.VMEM((_CHUNK, HEAD_DIM), jnp.float32),
            pltpu.SemaphoreType.DMA,
            pltpu.SemaphoreType.DMA,
        ],
    )
    def k(pos_hbm, cos_hbm, sin_hbm, outc_hbm, outs_hbm,
          idx_v, cbuf, sbuf, csem, ssem):
        wid = lax.axis_index("s") * _NC + lax.axis_index("c")
        base = wid * _B_PER_W
        pltpu.sync_copy(pos_hbm.at[pl.ds(base, _B_PER_W)], idx_v)
        for j in range(_NCHUNK):
            idxs = idx_v.at[pl.ds(j * _CHUNK, _CHUNK)]
            cc = pltpu.async_copy(cos_hbm.at[idxs], cbuf, csem)
            sc = pltpu.async_copy(sin_hbm.at[idxs], sbuf, ssem)
            row0 = base + j * _CHUNK
            cc.wait()
            pltpu.sync_copy(cbuf, outc_hbm.at[pl.ds(row0, _CHUNK)])
            sc.wait()
            pltpu.sync_copy(sbuf, outs_hbm.at[pl.ds(row0, _CHUNK)])

    return k(pos_flat, cos_cached, sin_cached)


def kernel(x, position_ids, cos_cached, sin_cached):
    b, s = position_ids.shape
    pos_flat = position_ids.reshape(-1)
    cos, sin = _rope_gather(pos_flat, cos_cached, sin_cached)
    return (cos.reshape(b, s, HEAD_DIM).astype(x.dtype),
            sin.reshape(b, s, HEAD_DIM).astype(x.dtype))


# speedup vs baseline: 1.7209x; 1.0588x over previous
import functools

import jax
import jax.numpy as jnp
from jax import lax
from jax.experimental import pallas as pl
from jax.experimental.pallas import tpu as pltpu
from jax.experimental.pallas import tpu_sc as plsc

HEAD_DIM = 256
B_TOTAL = 4 * 8192

_info = plsc.get_sparse_core_info()
_NC, _NS = _info.num_cores, _info.num_subcores
_NW = _NC * _NS
_B_PER_W = B_TOTAL // _NW
_CHUNK = 128
_NCHUNK = _B_PER_W // _CHUNK
_DEPTH = 3


def _rope_gather(pos_flat, cos_cached, sin_cached):
    mesh = plsc.VectorSubcoreMesh(core_axis_name="c", subcore_axis_name="s")

    @functools.partial(
        pl.kernel,
        mesh=mesh,
        out_type=[
            jax.ShapeDtypeStruct((B_TOTAL, HEAD_DIM), jnp.float32),
            jax.ShapeDtypeStruct((B_TOTAL, HEAD_DIM), jnp.float32),
        ],
        scratch_types=[
            pltpu.VMEM((_B_PER_W,), jnp.int32),
        ]
        + [pltpu.VMEM((_CHUNK, HEAD_DIM), jnp.float32)] * _DEPTH
        + [pltpu.SemaphoreType.DMA] * (2 * _DEPTH),
    )
    def k(pos_hbm, cos_hbm, sin_hbm, outc_hbm, outs_hbm, idx_v, *rest):
        bufs = list(rest[:_DEPTH])
        gsem = list(rest[_DEPTH:2 * _DEPTH])
        wsem = list(rest[2 * _DEPTH:])
        wid = lax.axis_index("s") * _NC + lax.axis_index("c")
        base = wid * _B_PER_W
        pltpu.sync_copy(pos_hbm.at[pl.ds(base, _B_PER_W)], idx_v)

        for tbl, out in ((cos_hbm, outc_hbm), (sin_hbm, outs_hbm)):
            wh = [None] * _DEPTH
            gh = [None] * _DEPTH
            for j in range(min(_DEPTH, _NCHUNK)):
                idxs = idx_v.at[pl.ds(j * _CHUNK, _CHUNK)]
                gh[j] = pltpu.async_copy(tbl.at[idxs], bufs[j], gsem[j])
            for j in range(_NCHUNK):
                b = j % _DEPTH
                gh[b].wait()
                row0 = base + j * _CHUNK
                wh[b] = pltpu.async_copy(
                    bufs[b], out.at[pl.ds(row0, _CHUNK)], wsem[b])
                jn = j + _DEPTH
                if jn < _NCHUNK:
                    wh[b].wait()
                    idxs = idx_v.at[pl.ds(jn * _CHUNK, _CHUNK)]
                    gh[b] = pltpu.async_copy(tbl.at[idxs], bufs[b], gsem[b])
            for j in range(max(0, _NCHUNK - _DEPTH), _NCHUNK):
                wh[j % _DEPTH].wait()

    return k(pos_flat, cos_cached, sin_cached)


def kernel(x, position_ids, cos_cached, sin_cached):
    b, s = position_ids.shape
    pos_flat = position_ids.reshape(-1)
    cos, sin = _rope_gather(pos_flat, cos_cached, sin_cached)
    return (cos.reshape(b, s, HEAD_DIM).astype(x.dtype),
            sin.reshape(b, s, HEAD_DIM).astype(x.dtype))
